# Initial kernel scaffold; baseline (speedup 1.0000x reference)
import jax
import jax.numpy as jnp
from jax.experimental import pallas as pl


def kernel(x, edge_index, W1, b1, W2, b2, Wp, bp):
    raise NotImplementedError("write your pallas kernel here")


# speedup vs baseline: 12.7533x; 12.7533x over previous
import functools

import jax
import jax.numpy as jnp
from jax import lax
from jax.experimental import pallas as pl
from jax.experimental.pallas import tpu as pltpu
from jax.experimental.pallas import tpu_sc as plsc

N = 10000
NP = 10016
E = 320000
EP = 321536
NT = 16
NC = 2
HALF = 5008
CH = 157
RT = 313
HS = 5120
C = 40
KHOPS = 8
BN = 2504

_mesh = plsc.VectorSubcoreMesh(core_axis_name="c", subcore_axis_name="s")
_sc_params = pltpu.CompilerParams(use_tc_tiling_on_sc=False)


def _fill_zero_rows(ref, nrows):
  def body(r, _):
    z = jnp.zeros((16,), jnp.float32)
    ref[r, pl.ds(0, 16)] = z
    ref[r, pl.ds(16, 16)] = z
    ref[r, pl.ds(24, 16)] = z
    return 0
  lax.fori_loop(0, nrows, body, 0)


def _pre_body(row_h, col_h, rowr_h, coll_h, hist_h,
              rbuf, cbuf, rrbuf, clbuf, hibuf, ones, hbuf, hacc):
  c = lax.axis_index("c")
  t = lax.axis_index("s")
  pltpu.sync_copy(row_h.at[t], rbuf)
  pltpu.sync_copy(col_h.at[t], cbuf)

  def zfill(l, _):
    hbuf[pl.ds(l * 16, 16)] = jnp.zeros((16,), jnp.float32)
    return 0
  lax.fori_loop(0, 20, zfill, 0)

  def ofill(l, _):
    ones[pl.ds(l * 16, 16)] = jnp.ones((16,), jnp.float32)
    return 0
  lax.fori_loop(0, 8, ofill, 0)

  pltpu.sync_copy(hbuf, hacc.at[pl.ds(t * 320, 320)])

  iota = lax.iota(jnp.int32, 16)
  base = c * HALF

  def edge_body(i, _):
    def lane_body(l, _):
      rv = rbuf[i, pl.ds(l * 16, 16)]
      cv = cbuf[i, pl.ds(l * 16, 16)]
      selfm = rv == cv
      dumpg = 10000 + iota
      dumpl = HALF + iota
      rr = jnp.where(selfm, dumpg, rv)
      lc = cv - base
      inr = lc.astype(jnp.uint32) < jnp.uint32(HALF)
      cl = jnp.where(inr, lc, dumpl)
      hi = jnp.where(selfm, dumpl, cl)
      rrbuf[i, pl.ds(l * 16, 16)] = rr
      clbuf[i, pl.ds(l * 16, 16)] = cl
      hibuf[i, pl.ds(l * 16, 16)] = hi
      return 0
    lax.fori_loop(0, 8, lane_body, 0)
    return 0
  lax.fori_loop(0, CH, edge_body, 0)

  plsc.subcore_barrier()

  def hist_body(i, _):
    pltpu.sync_copy(ones, hacc.at[hibuf.at[i]], add=True)
    return 0
  lax.fori_loop(0, CH, hist_body, 0)

  plsc.subcore_barrier()

  pltpu.sync_copy(hacc.at[pl.ds(t * 320, 320)], hbuf)
  pltpu.sync_copy(hbuf, hist_h.at[c, t])

  @pl.when(c == 0)
  def _():
    pltpu.sync_copy(rrbuf, rowr_h.at[t])
  pltpu.sync_copy(clbuf, coll_h.at[c, t])


_preprocess = pl.kernel(
    _pre_body,
    out_type=[
        jax.ShapeDtypeStruct((NT, CH, 128), jnp.int32),
        jax.ShapeDtypeStruct((NC, NT, CH, 128), jnp.int32),
        jax.ShapeDtypeStruct((NC, NT, 320), jnp.float32),
    ],
    mesh=_mesh,
    compiler_params=_sc_params,
    scratch_types=[
        pltpu.VMEM((CH, 128), jnp.int32),
        pltpu.VMEM((CH, 128), jnp.int32),
        pltpu.VMEM((CH, 128), jnp.int32),
        pltpu.VMEM((CH, 128), jnp.int32),
        pltpu.VMEM((CH, 128), jnp.int32),
        pltpu.VMEM((128,), jnp.float32),
        pltpu.VMEM((320,), jnp.float32),
        pltpu.VMEM_SHARED((HS,), jnp.float32),
    ],
)


def _hop_body(s_h, rowr_h, coll_h, dis_h, pred_h, snew_h,
              ridx, cidx, dbuf, abuf, sbuf, pbuf, s2buf, disb, zbuf, acc):
  c = lax.axis_index("c")
  t = lax.axis_index("s")

  _fill_zero_rows(zbuf, 314)
  pltpu.sync_copy(zbuf, acc.at[pl.ds(t * 314, 314)])
  pltpu.sync_copy(rowr_h.at[t], ridx)
  pltpu.sync_copy(coll_h.at[c, t], cidx)

  plsc.subcore_barrier()

  def chunk_body(i, _):
    pltpu.sync_copy(s_h.at[ridx.at[i]], dbuf)
    pltpu.sync_copy(dbuf, acc.at[cidx.at[i]], add=True)
    return 0
  lax.fori_loop(0, CH, chunk_body, 0)

  plsc.subcore_barrier()

  g0 = c * HALF + t * RT
  ga = (g0 // 8) * 8
  off = g0 - ga
  pltpu.sync_copy(dis_h.at[pl.ds(ga, 320)], disb.at[pl.ds(0, 320)])
  pltpu.sync_copy(acc.at[pl.ds(t * RT, RT)], abuf)
  pltpu.sync_copy(s_h.at[pl.ds(g0, RT)], sbuf)

  def comb_body(r, _):
    dvec = disb[pl.ds(r + off, 16)]
    dv = lax.broadcast(dvec[0], (16,))
    for offc in (0, 16, 24):
      a = abuf[r, pl.ds(offc, 16)]
      s = sbuf[r, pl.ds(offc, 16)]
      p = dv * (a + s)
      pbuf[r, pl.ds(offc, 16)] = p
      s2buf[r, pl.ds(offc, 16)] = dv * p
    return 0
  lax.fori_loop(0, RT, comb_body, 0)

  pltpu.sync_copy(pbuf, pred_h.at[pl.ds(g0, RT)])
  pltpu.sync_copy(s2buf, snew_h.at[pl.ds(g0, RT)])


_hop = pl.kernel(
    _hop_body,
    out_type=[
        jax.ShapeDtypeStruct((NP, C), jnp.float32),
        jax.ShapeDtypeStruct((NP, C), jnp.float32),
    ],
    mesh=_mesh,
    compiler_params=_sc_params,
    scratch_types=[
        pltpu.VMEM((CH, 128), jnp.int32),
        pltpu.VMEM((CH, 128), jnp.int32),
        pltpu.VMEM((128, C), jnp.float32),
        pltpu.VMEM((RT, C), jnp.float32),
        pltpu.VMEM((RT, C), jnp.float32),
        pltpu.VMEM((RT, C), jnp.float32),
        pltpu.VMEM((RT, C), jnp.float32),
        pltpu.VMEM((336,), jnp.float32),
        pltpu.VMEM((314, C), jnp.float32),
        pltpu.VMEM_SHARED((NT * 314, C), jnp.float32),
    ],
)


def _mlp_block(x_ref, w1_ref, b1_ref, w2_ref, b2_ref, hist_ref,
               pred0_ref, s0_ref, dis_ref):
  h = jnp.dot(x_ref[...], w1_ref[...], preferred_element_type=jnp.float32)
  h = jax.nn.relu(h + b1_ref[...])
  h = jnp.dot(h, w2_ref[...], preferred_element_type=jnp.float32)
  h = h + b2_ref[...]
  histb = hist_ref[0]
  rows = pl.program_id(0) * BN + lax.broadcasted_iota(jnp.int32, (BN, 1), 0)
  dis = lax.rsqrt(histb + 1.0)
  dis = jnp.where(rows < N, dis, 0.0)
  pred0_ref[...] = h
  s0_ref[...] = h * dis
  dis_ref[0] = dis


def _final_block(*refs):
  pred_refs = refs[:KHOPS + 1]
  wp_ref, bp_ref, out_ref = refs[KHOPS + 1], refs[KHOPS + 2], refs[KHOPS + 3]
  wp = wp_ref[...]
  bp = bp_ref[...]
  acc = jnp.zeros((BN, C), jnp.float32)
  for k in range(KHOPS + 1):
    pk = pred_refs[k][...]
    score = jax.nn.sigmoid(
        jnp.dot(pk, wp, preferred_element_type=jnp.float32) + bp)
    acc = acc + score * pk
  m = jnp.max(acc, axis=1, keepdims=True)
  z = acc - m
  out_ref[...] = z - jnp.log(jnp.sum(jnp.exp(z), axis=1, keepdims=True))


def kernel(x, edge_index, W1, b1, W2, b2, Wp, bp):
  row = edge_index[0].astype(jnp.int32)
  col = edge_index[1].astype(jnp.int32)
  padv = 10000 + (jnp.arange(EP - E, dtype=jnp.int32) % 16)
  row_p = jnp.concatenate([row, padv]).reshape(NT, CH, 128)
  col_p = jnp.concatenate([col, padv]).reshape(NT, CH, 128)
  x_pad = jnp.pad(x, ((0, NP - N), (0, 0)))

  rowr, coll, hist = _preprocess(row_p, col_p)
  hist_full = jnp.concatenate(
      [hist[0].reshape(HS)[:HALF], hist[1].reshape(HS)[:HALF]])
  hist3 = hist_full.reshape(4, BN, 1)

  pred0, s0, dis3 = pl.pallas_call(
      _mlp_block,
      grid=(4,),
      in_specs=[
          pl.BlockSpec((BN, 128), lambda i: (i, 0)),
          pl.BlockSpec((128, 256), lambda i: (0, 0)),
          pl.BlockSpec((1, 256), lambda i: (0, 0)),
          pl.BlockSpec((256, C), lambda i: (0, 0)),
          pl.BlockSpec((1, C), lambda i: (0, 0)),
          pl.BlockSpec((1, BN, 1), lambda i: (i, 0, 0)),
      ],
      out_specs=[
          pl.BlockSpec((BN, C), lambda i: (i, 0)),
          pl.BlockSpec((BN, C), lambda i: (i, 0)),
          pl.BlockSpec((1, BN, 1), lambda i: (i, 0, 0)),
      ],
      out_shape=[
          jax.ShapeDtypeStruct((NP, C), jnp.float32),
          jax.ShapeDtypeStruct((NP, C), jnp.float32),
          jax.ShapeDtypeStruct((4, BN, 1), jnp.float32),
      ],
  )(x_pad, W1, b1.reshape(1, 256), W2, b2.reshape(1, C), hist3)

  dis_full = dis3.reshape(NP)

  preds = [pred0]
  s = s0
  for _ in range(KHOPS):
    pred_k, s = _hop(s, rowr, coll, dis_full)
    preds.append(pred_k)

  out = pl.pallas_call(
      _final_block,
      grid=(4,),
      in_specs=[pl.BlockSpec((BN, C), lambda i: (i, 0))] * (KHOPS + 1)
      + [
          pl.BlockSpec((C, 1), lambda i: (0, 0)),
          pl.BlockSpec((1, 1), lambda i: (0, 0)),
      ],
      out_specs=pl.BlockSpec((BN, C), lambda i: (i, 0)),
      out_shape=jax.ShapeDtypeStruct((NP, C), jnp.float32),
  )(*preds, Wp, bp.reshape(1, 1))

  return out[:N]


# speedup vs baseline: 20.1472x; 1.5798x over previous
import functools

import jax
import jax.numpy as jnp
from jax import lax
from jax.experimental import pallas as pl
from jax.experimental.pallas import tpu as pltpu
from jax.experimental.pallas import tpu_sc as plsc

N = 10000
NP = 10016
E = 320000
EP = 323584
NT = 16
NC = 2
HALF = 5008
CH = 158
RT = 313
HS = 5120
C = 40
KHOPS = 8
BN = 2504

_mesh = plsc.VectorSubcoreMesh(core_axis_name="c", subcore_axis_name="s")
_sc_params = pltpu.CompilerParams(use_tc_tiling_on_sc=False)


def _fill_zero_rows(ref, nrows):
  def body(r, _):
    z = jnp.zeros((16,), jnp.float32)
    ref[r, pl.ds(0, 16)] = z
    ref[r, pl.ds(16, 16)] = z
    ref[r, pl.ds(24, 16)] = z
    return 0
  lax.fori_loop(0, nrows, body, 0)


def _pre_body(row_h, col_h, rowr_h, coll_h, hist_h,
              rbuf, cbuf, rrbuf, clbuf, hibuf, ones, hbuf, hacc):
  c = lax.axis_index("c")
  t = lax.axis_index("s")
  pltpu.sync_copy(row_h.at[t], rbuf)
  pltpu.sync_copy(col_h.at[t], cbuf)

  def zfill(l, _):
    hbuf[pl.ds(l * 16, 16)] = jnp.zeros((16,), jnp.float32)
    return 0
  lax.fori_loop(0, 20, zfill, 0)

  def ofill(l, _):
    ones[pl.ds(l * 16, 16)] = jnp.ones((16,), jnp.float32)
    return 0
  lax.fori_loop(0, 8, ofill, 0)

  pltpu.sync_copy(hbuf, hacc.at[pl.ds(t * 320, 320)])

  iota = lax.iota(jnp.int32, 16)
  base = c * HALF

  def edge_body(i, _):
    def lane_body(l, _):
      rv = rbuf[i, pl.ds(l * 16, 16)]
      cv = cbuf[i, pl.ds(l * 16, 16)]
      selfm = rv == cv
      dumpg = 10000 + iota
      dumpl = HALF + iota
      rr = jnp.where(selfm, dumpg, rv)
      lc = cv - base
      inr = lc.astype(jnp.uint32) < jnp.uint32(HALF)
      cl = jnp.where(inr, lc, dumpl)
      hi = jnp.where(selfm, dumpl, cl)
      rrbuf[i, pl.ds(l * 16, 16)] = rr
      clbuf[i, pl.ds(l * 16, 16)] = cl
      hibuf[i, pl.ds(l * 16, 16)] = hi
      return 0
    lax.fori_loop(0, 8, lane_body, 0)
    return 0
  lax.fori_loop(0, CH, edge_body, 0)

  plsc.subcore_barrier()

  def hist_body(i, _):
    pltpu.sync_copy(ones, hacc.at[hibuf.at[i]], add=True)
    return 0
  lax.fori_loop(0, CH, hist_body, 0)

  plsc.subcore_barrier()

  pltpu.sync_copy(hacc.at[pl.ds(t * 320, 320)], hbuf)
  pltpu.sync_copy(hbuf, hist_h.at[c, t])

  @pl.when(c == 0)
  def _():
    pltpu.sync_copy(rrbuf, rowr_h.at[t])
  pltpu.sync_copy(clbuf, coll_h.at[c, t])


_preprocess = pl.kernel(
    _pre_body,
    out_type=[
        jax.ShapeDtypeStruct((NT, CH, 128), jnp.int32),
        jax.ShapeDtypeStruct((NC, NT, CH, 128), jnp.int32),
        jax.ShapeDtypeStruct((NC, NT, 320), jnp.float32),
    ],
    mesh=_mesh,
    compiler_params=_sc_params,
    scratch_types=[
        pltpu.VMEM((CH, 128), jnp.int32),
        pltpu.VMEM((CH, 128), jnp.int32),
        pltpu.VMEM((CH, 128), jnp.int32),
        pltpu.VMEM((CH, 128), jnp.int32),
        pltpu.VMEM((CH, 128), jnp.int32),
        pltpu.VMEM((128,), jnp.float32),
        pltpu.VMEM((320,), jnp.float32),
        pltpu.VMEM_SHARED((HS,), jnp.float32),
    ],
)


def _hop_body(s_h, rowr_h, coll_h, dis_h, pred_h, snew_h,
              ridx, cidx, dbuf0, dbuf1, abuf, sbuf, pbuf, s2buf, disb, zbuf, acc,
              sem0, sem1):
  c = lax.axis_index("c")
  t = lax.axis_index("s")

  _fill_zero_rows(zbuf, 314)
  pltpu.sync_copy(zbuf, acc.at[pl.ds(t * 314, 314)])
  pltpu.sync_copy(rowr_h.at[t], ridx)
  pltpu.sync_copy(coll_h.at[c, t], cidx)

  plsc.subcore_barrier()

  pltpu.async_copy(s_h.at[ridx.at[0]], dbuf0, sem0)
  pltpu.async_copy(s_h.at[ridx.at[1]], dbuf1, sem1)

  def chunk_body(j, _):
    i0 = 2 * j
    pltpu.make_async_copy(s_h.at[ridx.at[i0]], dbuf0, sem0).wait()
    pltpu.sync_copy(dbuf0, acc.at[cidx.at[i0]], add=True)

    @pl.when(i0 + 2 < CH)
    def _():
      pltpu.async_copy(s_h.at[ridx.at[i0 + 2]], dbuf0, sem0)

    i1 = i0 + 1
    pltpu.make_async_copy(s_h.at[ridx.at[i1]], dbuf1, sem1).wait()
    pltpu.sync_copy(dbuf1, acc.at[cidx.at[i1]], add=True)

    @pl.when(i1 + 2 < CH)
    def _():
      pltpu.async_copy(s_h.at[ridx.at[i1 + 2]], dbuf1, sem1)
    return 0
  lax.fori_loop(0, CH // 2, chunk_body, 0)

  plsc.subcore_barrier()

  g0 = c * HALF + t * RT
  ga = (g0 // 8) * 8
  off = g0 - ga
  pltpu.sync_copy(dis_h.at[pl.ds(ga, 320)], disb.at[pl.ds(0, 320)])
  pltpu.sync_copy(acc.at[pl.ds(t * RT, RT)], abuf)
  pltpu.sync_copy(s_h.at[pl.ds(g0, RT)], sbuf)

  def comb_body(r, _):
    dvec = disb[pl.ds(r + off, 16)]
    dv = lax.broadcast(dvec[0], (16,))
    for offc in (0, 16, 24):
      a = abuf[r, pl.ds(offc, 16)]
      s = sbuf[r, pl.ds(offc, 16)]
      p = dv * (a + s)
      pbuf[r, pl.ds(offc, 16)] = p
      s2buf[r, pl.ds(offc, 16)] = dv * p
    return 0
  lax.fori_loop(0, RT, comb_body, 0)

  pltpu.sync_copy(pbuf, pred_h.at[pl.ds(g0, RT)])
  pltpu.sync_copy(s2buf, snew_h.at[pl.ds(g0, RT)])


_hop = pl.kernel(
    _hop_body,
    out_type=[
        jax.ShapeDtypeStruct((NP, C), jnp.float32),
        jax.ShapeDtypeStruct((NP, C), jnp.float32),
    ],
    mesh=_mesh,
    compiler_params=_sc_params,
    scratch_types=[
        pltpu.VMEM((CH, 128), jnp.int32),
        pltpu.VMEM((CH, 128), jnp.int32),
        pltpu.VMEM((128, C), jnp.float32),
        pltpu.VMEM((128, C), jnp.float32),
        pltpu.VMEM((RT, C), jnp.float32),
        pltpu.VMEM((RT, C), jnp.float32),
        pltpu.VMEM((RT, C), jnp.float32),
        pltpu.VMEM((RT, C), jnp.float32),
        pltpu.VMEM((336,), jnp.float32),
        pltpu.VMEM((314, C), jnp.float32),
        pltpu.VMEM_SHARED((NT * 314, C), jnp.float32),
        pltpu.SemaphoreType.DMA,
        pltpu.SemaphoreType.DMA,
    ],
)


def _mlp_block(x_ref, w1_ref, b1_ref, w2_ref, b2_ref, hist_ref,
               pred0_ref, s0_ref, dis_ref):
  h = jnp.dot(x_ref[...], w1_ref[...], preferred_element_type=jnp.float32)
  h = jax.nn.relu(h + b1_ref[...])
  h = jnp.dot(h, w2_ref[...], preferred_element_type=jnp.float32)
  h = h + b2_ref[...]
  histb = hist_ref[0]
  rows = pl.program_id(0) * BN + lax.broadcasted_iota(jnp.int32, (BN, 1), 0)
  dis = lax.rsqrt(histb + 1.0)
  dis = jnp.where(rows < N, dis, 0.0)
  pred0_ref[...] = h
  s0_ref[...] = h * dis
  dis_ref[0] = dis


def _final_block(*refs):
  pred_refs = refs[:KHOPS + 1]
  wp_ref, bp_ref, out_ref = refs[KHOPS + 1], refs[KHOPS + 2], refs[KHOPS + 3]
  wp = wp_ref[...]
  bp = bp_ref[...]
  acc = jnp.zeros((BN, C), jnp.float32)
  for k in range(KHOPS + 1):
    pk = pred_refs[k][...]
    score = jax.nn.sigmoid(
        jnp.dot(pk, wp, preferred_element_type=jnp.float32) + bp)
    acc = acc + score * pk
  m = jnp.max(acc, axis=1, keepdims=True)
  z = acc - m
  out_ref[...] = z - jnp.log(jnp.sum(jnp.exp(z), axis=1, keepdims=True))


def kernel(x, edge_index, W1, b1, W2, b2, Wp, bp):
  row = edge_index[0].astype(jnp.int32)
  col = edge_index[1].astype(jnp.int32)
  padv = 10000 + (jnp.arange(EP - E, dtype=jnp.int32) % 16)
  row_p = jnp.concatenate([row, padv]).reshape(NT, CH, 128)
  col_p = jnp.concatenate([col, padv]).reshape(NT, CH, 128)
  x_pad = jnp.pad(x, ((0, NP - N), (0, 0)))

  rowr, coll, hist = _preprocess(row_p, col_p)
  hist_full = jnp.concatenate(
      [hist[0].reshape(HS)[:HALF], hist[1].reshape(HS)[:HALF]])
  hist3 = hist_full.reshape(4, BN, 1)

  pred0, s0, dis3 = pl.pallas_call(
      _mlp_block,
      grid=(4,),
      in_specs=[
          pl.BlockSpec((BN, 128), lambda i: (i, 0)),
          pl.BlockSpec((128, 256), lambda i: (0, 0)),
          pl.BlockSpec((1, 256), lambda i: (0, 0)),
          pl.BlockSpec((256, C), lambda i: (0, 0)),
          pl.BlockSpec((1, C), lambda i: (0, 0)),
          pl.BlockSpec((1, BN, 1), lambda i: (i, 0, 0)),
      ],
      out_specs=[
          pl.BlockSpec((BN, C), lambda i: (i, 0)),
          pl.BlockSpec((BN, C), lambda i: (i, 0)),
          pl.BlockSpec((1, BN, 1), lambda i: (i, 0, 0)),
      ],
      out_shape=[
          jax.ShapeDtypeStruct((NP, C), jnp.float32),
          jax.ShapeDtypeStruct((NP, C), jnp.float32),
          jax.ShapeDtypeStruct((4, BN, 1), jnp.float32),
      ],
  )(x_pad, W1, b1.reshape(1, 256), W2, b2.reshape(1, C), hist3)

  dis_full = dis3.reshape(NP)

  preds = [pred0]
  s = s0
  for _ in range(KHOPS):
    pred_k, s = _hop(s, rowr, coll, dis_full)
    preds.append(pred_k)

  out = pl.pallas_call(
      _final_block,
      grid=(4,),
      in_specs=[pl.BlockSpec((BN, C), lambda i: (i, 0))] * (KHOPS + 1)
      + [
          pl.BlockSpec((C, 1), lambda i: (0, 0)),
          pl.BlockSpec((1, 1), lambda i: (0, 0)),
      ],
      out_specs=pl.BlockSpec((BN, C), lambda i: (i, 0)),
      out_shape=jax.ShapeDtypeStruct((NP, C), jnp.float32),
  )(*preds, Wp, bp.reshape(1, 1))

  return out[:N]


# speedup vs baseline: 20.1745x; 1.0014x over previous
import functools

import jax
import jax.numpy as jnp
from jax import lax
from jax.experimental import pallas as pl
from jax.experimental.pallas import tpu as pltpu
from jax.experimental.pallas import tpu_sc as plsc

N = 10000
NP = 10016
E = 320000
EP = 323584
NT = 16
NC = 2
HALF = 5008
CH = 158
RT = 313
HS = 6144
C = 40
KHOPS = 8
BN = 2504

_mesh = plsc.VectorSubcoreMesh(core_axis_name="c", subcore_axis_name="s")
_sc_params = pltpu.CompilerParams(use_tc_tiling_on_sc=False)


def _fill_zero_rows(ref, nrows):
  def body(r, _):
    z = jnp.zeros((16,), jnp.float32)
    ref[r, pl.ds(0, 16)] = z
    ref[r, pl.ds(16, 16)] = z
    ref[r, pl.ds(24, 16)] = z
    return 0
  lax.fori_loop(0, nrows, body, 0)


def _pre_body(row_h, col_h, rowr_h, coll_h, hist_h,
              rbuf, cbuf, rrbuf, clbuf, hibuf, ones, hbuf, hacc):
  c = lax.axis_index("c")
  t = lax.axis_index("s")
  pltpu.sync_copy(row_h.at[t], rbuf)
  pltpu.sync_copy(col_h.at[t], cbuf)

  def zfill(l, _):
    hbuf[pl.ds(l * 16, 16)] = jnp.zeros((16,), jnp.float32)
    return 0
  lax.fori_loop(0, 20, zfill, 0)

  def ofill(l, _):
    ones[pl.ds(l * 16, 16)] = jnp.ones((16,), jnp.float32)
    return 0
  lax.fori_loop(0, 8, ofill, 0)

  pltpu.sync_copy(hbuf, hacc.at[pl.ds(t * 320, 320)])

  iota = lax.iota(jnp.int32, 16)
  base = c * HALF

  def edge_body(i, _):
    def lane_body(l, _):
      rv = rbuf[i, pl.ds(l * 16, 16)]
      cv = cbuf[i, pl.ds(l * 16, 16)]
      selfm = rv == cv
      dumpg = 10000 + iota
      dumpl = 5024 + ((i * 128 + l * 16) & 1023) + iota
      rr = jnp.where(selfm, dumpg, rv)
      lc = cv - base
      inr = lc.astype(jnp.uint32) < jnp.uint32(HALF)
      cl = jnp.where(inr, lc, dumpl)
      hi = jnp.where(selfm, dumpl, cl)
      rrbuf[i, pl.ds(l * 16, 16)] = rr
      clbuf[i, pl.ds(l * 16, 16)] = cl
      hibuf[i, pl.ds(l * 16, 16)] = hi
      return 0
    lax.fori_loop(0, 8, lane_body, 0)
    return 0
  lax.fori_loop(0, CH, edge_body, 0)

  plsc.subcore_barrier()

  def hist_body(i, _):
    pltpu.sync_copy(ones, hacc.at[hibuf.at[i]], add=True)
    return 0
  lax.fori_loop(0, CH, hist_body, 0)

  plsc.subcore_barrier()

  pltpu.sync_copy(hacc.at[pl.ds(t * 320, 320)], hbuf)
  pltpu.sync_copy(hbuf, hist_h.at[c, t])

  @pl.when(c == 0)
  def _():
    pltpu.sync_copy(rrbuf, rowr_h.at[t])
  pltpu.sync_copy(clbuf, coll_h.at[c, t])


_preprocess = pl.kernel(
    _pre_body,
    out_type=[
        jax.ShapeDtypeStruct((NT, CH, 128), jnp.int32),
        jax.ShapeDtypeStruct((NC, NT, CH, 128), jnp.int32),
        jax.ShapeDtypeStruct((NC, NT, 320), jnp.float32),
    ],
    mesh=_mesh,
    compiler_params=_sc_params,
    scratch_types=[
        pltpu.VMEM((CH, 128), jnp.int32),
        pltpu.VMEM((CH, 128), jnp.int32),
        pltpu.VMEM((CH, 128), jnp.int32),
        pltpu.VMEM((CH, 128), jnp.int32),
        pltpu.VMEM((CH, 128), jnp.int32),
        pltpu.VMEM((128,), jnp.float32),
        pltpu.VMEM((320,), jnp.float32),
        pltpu.VMEM_SHARED((HS,), jnp.float32),
    ],
)


def _hop_body(s_h, rowr_h, coll_h, dis_h, pred_h, snew_h,
              ridx, cidx, dbuf0, dbuf1, abuf, sbuf, pbuf, s2buf, disb, zbuf, acc,
              sem0, sem1):
  c = lax.axis_index("c")
  t = lax.axis_index("s")

  _fill_zero_rows(zbuf, 314)
  pltpu.sync_copy(zbuf, acc.at[pl.ds(t * 314, 314)])
  pltpu.sync_copy(rowr_h.at[t], ridx)
  pltpu.sync_copy(coll_h.at[c, t], cidx)

  plsc.subcore_barrier()

  pltpu.async_copy(s_h.at[ridx.at[0]], dbuf0, sem0)
  pltpu.async_copy(s_h.at[ridx.at[1]], dbuf1, sem1)

  def chunk_body(j, _):
    i0 = 2 * j
    pltpu.make_async_copy(s_h.at[ridx.at[i0]], dbuf0, sem0).wait()
    pltpu.sync_copy(dbuf0, acc.at[cidx.at[i0]], add=True)

    @pl.when(i0 + 2 < CH)
    def _():
      pltpu.async_copy(s_h.at[ridx.at[i0 + 2]], dbuf0, sem0)

    i1 = i0 + 1
    pltpu.make_async_copy(s_h.at[ridx.at[i1]], dbuf1, sem1).wait()
    pltpu.sync_copy(dbuf1, acc.at[cidx.at[i1]], add=True)

    @pl.when(i1 + 2 < CH)
    def _():
      pltpu.async_copy(s_h.at[ridx.at[i1 + 2]], dbuf1, sem1)
    return 0
  lax.fori_loop(0, CH // 2, chunk_body, 0)

  plsc.subcore_barrier()

  g0 = c * HALF + t * RT
  ga = (g0 // 8) * 8
  off = g0 - ga
  pltpu.sync_copy(dis_h.at[pl.ds(ga, 320)], disb.at[pl.ds(0, 320)])
  pltpu.sync_copy(acc.at[pl.ds(t * RT, RT)], abuf)
  pltpu.sync_copy(s_h.at[pl.ds(g0, RT)], sbuf)

  def comb_body(r, _):
    dvec = disb[pl.ds(r + off, 16)]
    dv = lax.broadcast(dvec[0], (16,))
    for offc in (0, 16, 24):
      a = abuf[r, pl.ds(offc, 16)]
      s = sbuf[r, pl.ds(offc, 16)]
      p = dv * (a + s)
      pbuf[r, pl.ds(offc, 16)] = p
      s2buf[r, pl.ds(offc, 16)] = dv * p
    return 0
  lax.fori_loop(0, RT, comb_body, 0)

  pltpu.sync_copy(pbuf, pred_h.at[pl.ds(g0, RT)])
  pltpu.sync_copy(s2buf, snew_h.at[pl.ds(g0, RT)])


_hop = pl.kernel(
    _hop_body,
    out_type=[
        jax.ShapeDtypeStruct((NP, C), jnp.float32),
        jax.ShapeDtypeStruct((NP, C), jnp.float32),
    ],
    mesh=_mesh,
    compiler_params=_sc_params,
    scratch_types=[
        pltpu.VMEM((CH, 128), jnp.int32),
        pltpu.VMEM((CH, 128), jnp.int32),
        pltpu.VMEM((128, C), jnp.float32),
        pltpu.VMEM((128, C), jnp.float32),
        pltpu.VMEM((RT, C), jnp.float32),
        pltpu.VMEM((RT, C), jnp.float32),
        pltpu.VMEM((RT, C), jnp.float32),
        pltpu.VMEM((RT, C), jnp.float32),
        pltpu.VMEM((336,), jnp.float32),
        pltpu.VMEM((314, C), jnp.float32),
        pltpu.VMEM_SHARED((6144, C), jnp.float32),
        pltpu.SemaphoreType.DMA,
        pltpu.SemaphoreType.DMA,
    ],
)


def _mlp_block(x_ref, w1_ref, b1_ref, w2_ref, b2_ref, hist_ref,
               pred0_ref, s0_ref, dis_ref):
  h = jnp.dot(x_ref[...], w1_ref[...], preferred_element_type=jnp.float32)
  h = jax.nn.relu(h + b1_ref[...])
  h = jnp.dot(h, w2_ref[...], preferred_element_type=jnp.float32)
  h = h + b2_ref[...]
  histb = hist_ref[0]
  rows = pl.program_id(0) * BN + lax.broadcasted_iota(jnp.int32, (BN, 1), 0)
  dis = lax.rsqrt(histb + 1.0)
  dis = jnp.where(rows < N, dis, 0.0)
  pred0_ref[...] = h
  s0_ref[...] = h * dis
  dis_ref[0] = dis


def _final_block(*refs):
  pred_refs = refs[:KHOPS + 1]
  wp_ref, bp_ref, out_ref = refs[KHOPS + 1], refs[KHOPS + 2], refs[KHOPS + 3]
  wp = wp_ref[...]
  bp = bp_ref[...]
  acc = jnp.zeros((BN, C), jnp.float32)
  for k in range(KHOPS + 1):
    pk = pred_refs[k][...]
    score = jax.nn.sigmoid(
        jnp.dot(pk, wp, preferred_element_type=jnp.float32) + bp)
    acc = acc + score * pk
  m = jnp.max(acc, axis=1, keepdims=True)
  z = acc - m
  out_ref[...] = z - jnp.log(jnp.sum(jnp.exp(z), axis=1, keepdims=True))


def kernel(x, edge_index, W1, b1, W2, b2, Wp, bp):
  row = edge_index[0].astype(jnp.int32)
  col = edge_index[1].astype(jnp.int32)
  padv = 10000 + (jnp.arange(EP - E, dtype=jnp.int32) % 16)
  row_p = jnp.concatenate([row, padv]).reshape(NT, CH, 128)
  col_p = jnp.concatenate([col, padv]).reshape(NT, CH, 128)
  x_pad = jnp.pad(x, ((0, NP - N), (0, 0)))

  rowr, coll, hist = _preprocess(row_p, col_p)
  hist_full = jnp.concatenate(
      [hist[0].reshape(NT * 320)[:HALF], hist[1].reshape(NT * 320)[:HALF]])
  hist3 = hist_full.reshape(4, BN, 1)

  pred0, s0, dis3 = pl.pallas_call(
      _mlp_block,
      grid=(4,),
      in_specs=[
          pl.BlockSpec((BN, 128), lambda i: (i, 0)),
          pl.BlockSpec((128, 256), lambda i: (0, 0)),
          pl.BlockSpec((1, 256), lambda i: (0, 0)),
          pl.BlockSpec((256, C), lambda i: (0, 0)),
          pl.BlockSpec((1, C), lambda i: (0, 0)),
          pl.BlockSpec((1, BN, 1), lambda i: (i, 0, 0)),
      ],
      out_specs=[
          pl.BlockSpec((BN, C), lambda i: (i, 0)),
          pl.BlockSpec((BN, C), lambda i: (i, 0)),
          pl.BlockSpec((1, BN, 1), lambda i: (i, 0, 0)),
      ],
      out_shape=[
          jax.ShapeDtypeStruct((NP, C), jnp.float32),
          jax.ShapeDtypeStruct((NP, C), jnp.float32),
          jax.ShapeDtypeStruct((4, BN, 1), jnp.float32),
      ],
  )(x_pad, W1, b1.reshape(1, 256), W2, b2.reshape(1, C), hist3)

  dis_full = dis3.reshape(NP)

  preds = [pred0]
  s = s0
  for _ in range(KHOPS):
    pred_k, s = _hop(s, rowr, coll, dis_full)
    preds.append(pred_k)

  out = pl.pallas_call(
      _final_block,
      grid=(4,),
      in_specs=[pl.BlockSpec((BN, C), lambda i: (i, 0))] * (KHOPS + 1)
      + [
          pl.BlockSpec((C, 1), lambda i: (0, 0)),
          pl.BlockSpec((1, 1), lambda i: (0, 0)),
      ],
      out_specs=pl.BlockSpec((BN, C), lambda i: (i, 0)),
      out_shape=jax.ShapeDtypeStruct((NP, C), jnp.float32),
  )(*preds, Wp, bp.reshape(1, 1))

  return out[:N]


# speedup vs baseline: 24.8479x; 1.2316x over previous
import functools

import jax
import jax.numpy as jnp
from jax import lax
from jax.experimental import pallas as pl
from jax.experimental.pallas import tpu as pltpu
from jax.experimental.pallas import tpu_sc as plsc

N = 10000
NP = 10016
E = 320000
EP = 327680
NT = 16
NC = 2
CH = 80
RS = 626
ACC_ROWS = 11072
HS = 11264
C = 40
KHOPS = 8
BN = 2504

_mesh = plsc.VectorSubcoreMesh(core_axis_name="c", subcore_axis_name="s")
_sc_params = pltpu.CompilerParams(use_tc_tiling_on_sc=False)


def _fill_zero_rows(ref, nrows):
  def body(r, _):
    z = jnp.zeros((16,), jnp.float32)
    ref[r, pl.ds(0, 16)] = z
    ref[r, pl.ds(16, 16)] = z
    ref[r, pl.ds(24, 16)] = z
    return 0
  lax.fori_loop(0, nrows, body, 0)


def _pre_body(row_h, col_h, rowr_h, coll_h, hist_h,
              rbuf, cbuf, rrbuf, clbuf, ones, hbuf, hacc):
  c = lax.axis_index("c")
  t = lax.axis_index("s")
  pltpu.sync_copy(row_h.at[c, t], rbuf)
  pltpu.sync_copy(col_h.at[c, t], cbuf)

  def zfill(l, _):
    hbuf[pl.ds(l * 16, 16)] = jnp.zeros((16,), jnp.float32)
    return 0
  lax.fori_loop(0, 44, zfill, 0)

  def ofill(l, _):
    ones[pl.ds(l * 16, 16)] = jnp.ones((16,), jnp.float32)
    return 0
  lax.fori_loop(0, 8, ofill, 0)

  pltpu.sync_copy(hbuf.at[pl.ds(0, 704)], hacc.at[pl.ds(t * 704, 704)])

  iota = lax.iota(jnp.int32, 16)

  def edge_body(i, _):
    def lane_body(l, _):
      rv = rbuf[i, pl.ds(l * 16, 16)]
      cv = cbuf[i, pl.ds(l * 16, 16)]
      selfm = rv == cv
      dumpg = 10000 + iota
      dumpl = 10016 + ((i * 128 + l * 16) & 1023) + iota
      rr = jnp.where(selfm, dumpg, rv)
      cl = jnp.where(selfm, dumpl, cv)
      rrbuf[i, pl.ds(l * 16, 16)] = rr
      clbuf[i, pl.ds(l * 16, 16)] = cl
      return 0
    lax.fori_loop(0, 8, lane_body, 0)
    return 0
  lax.fori_loop(0, CH, edge_body, 0)

  plsc.subcore_barrier()

  def hist_body(i, _):
    pltpu.sync_copy(ones, hacc.at[clbuf.at[i]], add=True)
    return 0
  lax.fori_loop(0, CH, hist_body, 0)

  plsc.subcore_barrier()

  pltpu.sync_copy(hacc.at[pl.ds(t * 640, 640)], hbuf.at[pl.ds(0, 640)])
  pltpu.sync_copy(hbuf.at[pl.ds(0, 640)], hist_h.at[c, t])

  pltpu.sync_copy(rrbuf, rowr_h.at[c, t])
  pltpu.sync_copy(clbuf, coll_h.at[c, t])


_preprocess = pl.kernel(
    _pre_body,
    out_type=[
        jax.ShapeDtypeStruct((NC, NT, CH, 128), jnp.int32),
        jax.ShapeDtypeStruct((NC, NT, CH, 128), jnp.int32),
        jax.ShapeDtypeStruct((NC, NT, 640), jnp.float32),
    ],
    mesh=_mesh,
    compiler_params=_sc_params,
    scratch_types=[
        pltpu.VMEM((CH, 128), jnp.int32),
        pltpu.VMEM((CH, 128), jnp.int32),
        pltpu.VMEM((CH, 128), jnp.int32),
        pltpu.VMEM((CH, 128), jnp.int32),
        pltpu.VMEM((128,), jnp.float32),
        pltpu.VMEM((704,), jnp.float32),
        pltpu.VMEM_SHARED((HS,), jnp.float32),
    ],
)


def _hop_body(s_h, rowr_h, coll_h, partial_h,
              ridx, cidx, dbuf0, dbuf1, zbuf, obuf, acc, sem0, sem1):
  c = lax.axis_index("c")
  t = lax.axis_index("s")

  _fill_zero_rows(zbuf, RS)
  pltpu.sync_copy(zbuf, acc.at[pl.ds(t * RS, RS)])
  pltpu.sync_copy(zbuf.at[pl.ds(0, 66)], acc.at[pl.ds(NP + t * 66, 66)])
  pltpu.sync_copy(rowr_h.at[c, t], ridx)
  pltpu.sync_copy(coll_h.at[c, t], cidx)

  plsc.subcore_barrier()

  pltpu.async_copy(s_h.at[ridx.at[0]], dbuf0, sem0)
  pltpu.async_copy(s_h.at[ridx.at[1]], dbuf1, sem1)

  def chunk_body(j, _):
    i0 = 2 * j
    pltpu.make_async_copy(s_h.at[ridx.at[i0]], dbuf0, sem0).wait()
    pltpu.sync_copy(dbuf0, acc.at[cidx.at[i0]], add=True)

    @pl.when(i0 + 2 < CH)
    def _():
      pltpu.async_copy(s_h.at[ridx.at[i0 + 2]], dbuf0, sem0)

    i1 = i0 + 1
    pltpu.make_async_copy(s_h.at[ridx.at[i1]], dbuf1, sem1).wait()
    pltpu.sync_copy(dbuf1, acc.at[cidx.at[i1]], add=True)

    @pl.when(i1 + 2 < CH)
    def _():
      pltpu.async_copy(s_h.at[ridx.at[i1 + 2]], dbuf1, sem1)
    return 0
  lax.fori_loop(0, CH // 2, chunk_body, 0)

  plsc.subcore_barrier()

  pltpu.sync_copy(acc.at[pl.ds(t * RS, RS)], obuf)
  pltpu.sync_copy(obuf, partial_h.at[c, pl.ds(t * RS, RS)])


_hop = pl.kernel(
    _hop_body,
    out_type=[
        jax.ShapeDtypeStruct((NC, NP, C), jnp.float32),
    ],
    mesh=_mesh,
    compiler_params=_sc_params,
    scratch_types=[
        pltpu.VMEM((CH, 128), jnp.int32),
        pltpu.VMEM((CH, 128), jnp.int32),
        pltpu.VMEM((128, C), jnp.float32),
        pltpu.VMEM((128, C), jnp.float32),
        pltpu.VMEM((RS, C), jnp.float32),
        pltpu.VMEM((RS, C), jnp.float32),
        pltpu.VMEM_SHARED((ACC_ROWS, C), jnp.float32),
        pltpu.SemaphoreType.DMA,
        pltpu.SemaphoreType.DMA,
    ],
)


def _mlp_block(x_ref, w1_ref, b1_ref, w2_ref, b2_ref, ha_ref, hb_ref,
               pred0_ref, s0_ref, dis_ref):
  h = jnp.dot(x_ref[...], w1_ref[...], preferred_element_type=jnp.float32)
  h = jax.nn.relu(h + b1_ref[...])
  h = jnp.dot(h, w2_ref[...], preferred_element_type=jnp.float32)
  h = h + b2_ref[...]
  histb = ha_ref[0] + hb_ref[0]
  rows = pl.program_id(0) * BN + lax.broadcasted_iota(jnp.int32, (BN, 1), 0)
  dis = lax.rsqrt(histb + 1.0)
  dis = jnp.where(rows < N, dis, 0.0)
  pred0_ref[...] = h
  s0_ref[...] = h * dis
  dis_ref[0] = dis


def _comb_block(p0_ref, p1_ref, s_ref, dis_ref, pred_ref, snew_ref):
  dis = dis_ref[0]
  pred = dis * (p0_ref[0] + p1_ref[0] + s_ref[...])
  pred_ref[...] = pred
  snew_ref[...] = dis * pred


def _final_block(*refs):
  pred_refs = refs[:KHOPS + 1]
  wp_ref, bp_ref, out_ref = refs[KHOPS + 1], refs[KHOPS + 2], refs[KHOPS + 3]
  wp = wp_ref[...]
  bp = bp_ref[...]
  acc = jnp.zeros((BN, C), jnp.float32)
  for k in range(KHOPS + 1):
    pk = pred_refs[k][...]
    score = jax.nn.sigmoid(
        jnp.dot(pk, wp, preferred_element_type=jnp.float32) + bp)
    acc = acc + score * pk
  m = jnp.max(acc, axis=1, keepdims=True)
  z = acc - m
  out_ref[...] = z - jnp.log(jnp.sum(jnp.exp(z), axis=1, keepdims=True))


def kernel(x, edge_index, W1, b1, W2, b2, Wp, bp):
  row = edge_index[0].astype(jnp.int32)
  col = edge_index[1].astype(jnp.int32)
  padv = 10000 + (jnp.arange(EP - E, dtype=jnp.int32) % 16)
  row_p = jnp.concatenate([row, padv]).reshape(NC, NT, CH, 128)
  col_p = jnp.concatenate([col, padv]).reshape(NC, NT, CH, 128)
  x_pad = jnp.pad(x, ((0, NP - N), (0, 0)))

  rowr, coll, hist = _preprocess(row_p, col_p)
  ha = hist[0].reshape(NT * 640)[:NP].reshape(4, BN, 1)
  hb = hist[1].reshape(NT * 640)[:NP].reshape(4, BN, 1)

  pred0, s0, dis3 = pl.pallas_call(
      _mlp_block,
      grid=(4,),
      in_specs=[
          pl.BlockSpec((BN, 128), lambda i: (i, 0)),
          pl.BlockSpec((128, 256), lambda i: (0, 0)),
          pl.BlockSpec((1, 256), lambda i: (0, 0)),
          pl.BlockSpec((256, C), lambda i: (0, 0)),
          pl.BlockSpec((1, C), lambda i: (0, 0)),
          pl.BlockSpec((1, BN, 1), lambda i: (i, 0, 0)),
          pl.BlockSpec((1, BN, 1), lambda i: (i, 0, 0)),
      ],
      out_specs=[
          pl.BlockSpec((BN, C), lambda i: (i, 0)),
          pl.BlockSpec((BN, C), lambda i: (i, 0)),
          pl.BlockSpec((1, BN, 1), lambda i: (i, 0, 0)),
      ],
      out_shape=[
          jax.ShapeDtypeStruct((NP, C), jnp.float32),
          jax.ShapeDtypeStruct((NP, C), jnp.float32),
          jax.ShapeDtypeStruct((4, BN, 1), jnp.float32),
      ],
  )(x_pad, W1, b1.reshape(1, 256), W2, b2.reshape(1, C), ha, hb)

  comb = pl.pallas_call(
      _comb_block,
      grid=(4,),
      in_specs=[
          pl.BlockSpec((1, BN, C), lambda i: (0, i, 0)),
          pl.BlockSpec((1, BN, C), lambda i: (1, i, 0)),
          pl.BlockSpec((BN, C), lambda i: (i, 0)),
          pl.BlockSpec((1, BN, 1), lambda i: (i, 0, 0)),
      ],
      out_specs=[
          pl.BlockSpec((BN, C), lambda i: (i, 0)),
          pl.BlockSpec((BN, C), lambda i: (i, 0)),
      ],
      out_shape=[
          jax.ShapeDtypeStruct((NP, C), jnp.float32),
          jax.ShapeDtypeStruct((NP, C), jnp.float32),
      ],
  )

  preds = [pred0]
  s = s0
  for _ in range(KHOPS):
    (partial,) = _hop(s, rowr, coll)
    pred_k, s = comb(partial, partial, s, dis3)
    preds.append(pred_k)

  out = pl.pallas_call(
      _final_block,
      grid=(4,),
      in_specs=[pl.BlockSpec((BN, C), lambda i: (i, 0))] * (KHOPS + 1)
      + [
          pl.BlockSpec((C, 1), lambda i: (0, 0)),
          pl.BlockSpec((1, 1), lambda i: (0, 0)),
      ],
      out_specs=pl.BlockSpec((BN, C), lambda i: (i, 0)),
      out_shape=jax.ShapeDtypeStruct((NP, C), jnp.float32),
  )(*preds, Wp, bp.reshape(1, 1))

  return out[:N]
